# baseline (speedup 1.0000x reference)
import functools

import jax
import jax.numpy as jnp
from jax.experimental import pallas as pl
from jax.experimental.pallas import tpu as pltpu

FRAC = 0.5

_KS1 = 42
_KS2 = 0x1BD11BDA ^ 42
_R1 = (13, 15, 26, 6)
_R2 = (17, 29, 16, 24)
_C23 = 15.942385152878742


def _rotl(v, r):
    return (v << r) | (v >> (32 - r))


def _threefry(x0, x1):
    for r in _R1:
        x0 = x0 + x1
        x1 = _rotl(x1, r) ^ x0
    x0 = x0 + _KS1
    x1 = x1 + (_KS2 + 1)
    for r in _R2:
        x0 = x0 + x1
        x1 = _rotl(x1, r) ^ x0
    x0 = x0 + _KS2
    x1 = x1 + 2
    for r in _R1:
        x0 = x0 + x1
        x1 = _rotl(x1, r) ^ x0
    x1 = x1 + (_KS1 + 3)
    for r in _R2:
        x0 = x0 + x1
        x1 = _rotl(x1, r) ^ x0
    x0 = x0 + _KS1
    x1 = x1 + (_KS2 + 4)
    for r in _R1:
        x0 = x0 + x1
        x1 = _rotl(x1, r) ^ x0
    x0 = x0 + _KS2
    x1 = x1 + 5
    return x0 ^ x1


def _sap_body(x_ref, o_ref, w_ref, *, nb, keep, shift_n, shift_b, cr):
    rows = x_ref.shape[1]
    nch = rows // cr
    chunk = cr * 128

    xr = x_ref[0]
    absx = jnp.abs(xr)
    s = jnp.sum(absx)
    prob = absx / s
    scale = jnp.maximum(1.0 - jnp.power(1.0 - prob, jnp.float32(keep)), 0.0001)
    w_ref[...] = 1.0 / jnp.maximum(absx, 1e-30)
    o_ref[0] = xr / scale

    b = pl.program_id(0)
    bu = jnp.uint32(b)
    if (nb - 1).bit_length() + shift_b > 32:
        hi = bu >> (32 - shift_b)
        lo_b = (bu & ((1 << (32 - shift_b)) - 1)) << shift_b
    else:
        hi = jnp.uint32(0)
        lo_b = bu << shift_b

    sub_i = jax.lax.broadcasted_iota(jnp.uint32, (cr, 128), 0)
    lane_i = jax.lax.broadcasted_iota(jnp.uint32, (cr, 128), 1)
    n_loc = sub_i * 128 + lane_i
    sub8 = jax.lax.broadcasted_iota(jnp.int32, (8, 128), 0)
    lane8 = jax.lax.broadcasted_iota(jnp.int32, (8, 128), 1)

    def k_body(k, carry):
        x1_base = lo_b + (jnp.uint32(k) << shift_n) + _KS1

        run_min = jnp.full((cr, 128), jnp.inf, jnp.float32)
        run_idx = jnp.zeros((cr, 128), jnp.int32)
        for c in range(nch):
            n_c = n_loc + jnp.uint32(c * chunk)
            bits = _threefry(hi, n_c + x1_base)
            m = (bits >> 9).astype(jnp.int32)
            u = m.astype(jnp.float32) * jnp.float32(2.0 ** -23)
            e = 0.0 - jnp.log(u)
            w = w_ref[c * cr:(c + 1) * cr, :]
            ew = e * w
            upd = ew < run_min
            run_min = jnp.where(upd, ew, run_min)
            run_idx = jnp.where(upd, n_c.astype(jnp.int32), run_idx)
        mval = jnp.min(run_min)
        idx = jnp.min(jnp.where(run_min == mval, run_idx, jnp.int32(2**30)))
        r8 = (idx >> 10) << 3
        sub = (idx >> 7) & 7
        col = idx & 127
        blk = o_ref[0, pl.ds(r8, 8), :]
        hit = (sub8 == sub) & (lane8 == col)
        o_ref[0, pl.ds(r8, 8), :] = jnp.where(hit, 0.0, blk)
        return carry

    jax.lax.fori_loop(0, keep, k_body, 0)


def kernel(x):
    B, N = x.shape
    keep = int(N * FRAC)
    rows = N // 128
    shift_n = N.bit_length() - 1
    shift_b = shift_n + keep.bit_length() - 1
    xr = x.reshape(B, rows, 128)
    body = functools.partial(_sap_body, nb=B, keep=keep, shift_n=shift_n,
                             shift_b=shift_b, cr=min(32, rows))
    out = pl.pallas_call(
        body,
        grid=(B,),
        in_specs=[pl.BlockSpec((1, rows, 128), lambda i: (i, 0, 0))],
        out_specs=pl.BlockSpec((1, rows, 128), lambda i: (i, 0, 0)),
        out_shape=jax.ShapeDtypeStruct((B, rows, 128), x.dtype),
        scratch_shapes=[pltpu.VMEM((rows, 128), jnp.float32)],
        compiler_params=pltpu.CompilerParams(
            dimension_semantics=("arbitrary",)),
    )(xr)
    return out.reshape(B, N)
